# baseline (speedup 1.0000x reference)
import functools

import jax
import jax.numpy as jnp
from jax import lax
from jax.experimental import pallas as pl
from jax.experimental.pallas import tpu as pltpu
from jax.experimental.pallas import tpu_sc as plsc

N = 10000
E = 320000
B = 64
ND = 128
ED = 16
GD = 64

NC = 2
NS = 16
NW = NC * NS

PT = 10240
EP = NW * PT
KI = 8
STEP = KI * 128
HALF = STEP // 2
QTR = STEP // 4
ACCR = 10240
RPT = ACCR // NS

BE = 2560
GE = EP // BE
REAL = E // BE
BN = 2000
GN = N // BN

_sc_cache = {}


def _mesh():
    return plsc.VectorSubcoreMesh(
        core_axis_name="c", subcore_axis_name="s",
        num_cores=NC, num_subcores=NS,
    )


def _stage_x(x_hbm, xsp, s):
    @pl.when(s < NS - 1)
    def _():
        pltpu.sync_copy(x_hbm.at[pl.ds(s * 640, 640)],
                        xsp.at[pl.ds(s * 640, 640)])

    @pl.when(s == NS - 1)
    def _():
        pltpu.sync_copy(x_hbm.at[pl.ds(9600, 400)],
                        xsp.at[pl.ds(9600, 400)])


def _row_phase(idx2, out_hbm, xsp, idx_v, rows_v, gsem, osem, rowbase, ebase):
    def group(g, _):
        pltpu.sync_copy(idx2.at[pl.ds(rowbase + g * KI, KI)], idx_v)
        gd = [None, None]
        outs = [None, None]
        for c8 in range(KI):
            b = c8 % 2
            if outs[b] is not None:
                outs[b].wait()
            gd[b] = pltpu.async_copy(
                xsp.at[idx_v.at[c8]], rows_v.at[b], gsem[b]
            )
            if c8 >= 1:
                pb = 1 - b
                gd[pb].wait()
                outs[pb] = pltpu.async_copy(
                    rows_v.at[pb],
                    out_hbm.at[pl.ds(ebase + g * STEP + (c8 - 1) * 128, 128)],
                    osem[pb],
                )
        lb = (KI - 1) % 2
        gd[lb].wait()
        outs[lb] = pltpu.async_copy(
            rows_v.at[lb],
            out_hbm.at[pl.ds(ebase + g * STEP + (KI - 1) * 128, 128)],
            osem[lb],
        )
        outs[0].wait()
        outs[1].wait()
        return 0

    lax.fori_loop(0, PT // STEP, group, 0)


def _sc_gather1_body(x_hbm, src2, xsrc_out, xsp, idx_v, rows_v,
                     gsem0, gsem1, osem0, osem1):
    c = lax.axis_index("c")
    s = lax.axis_index("s")
    wid = s * NC + c
    _stage_x(x_hbm, xsp, s)
    plsc.subcore_barrier()
    _row_phase(src2, xsrc_out, xsp, idx_v, rows_v, [gsem0, gsem1],
               [osem0, osem1], wid * (PT // 128), wid * PT)


def _sc_gather2_body(x_hbm, dst2, srcb2, batch_ext, xdst_out, bsrc_out,
                     xsp, bsp, idx_v, rows_v, bvals_v,
                     gsem0, gsem1, osem0, osem1):
    c = lax.axis_index("c")
    s = lax.axis_index("s")
    wid = s * NC + c
    rowbase = wid * (PT // 128)
    ebase = wid * PT
    _stage_x(x_hbm, xsp, s)

    @pl.when(s == 0)
    def _():
        pltpu.sync_copy(batch_ext, bsp)

    plsc.subcore_barrier()
    _row_phase(dst2, xdst_out, xsp, idx_v, rows_v, [gsem0, gsem1],
               [osem0, osem1], rowbase, ebase)

    def bgroup(g, _):
        pltpu.sync_copy(srcb2.at[pl.ds(rowbase + g * KI, KI)], idx_v)
        descs = [
            pltpu.async_copy(
                bsp.at[idx_v.at[j]],
                bvals_v.at[pl.ds(j * 128, 128)],
                gsem0,
            )
            for j in range(KI)
        ]
        for d in descs:
            d.wait()
        pltpu.sync_copy(bvals_v, bsrc_out.at[pl.ds(ebase + g * STEP, STEP)])
        return 0

    lax.fori_loop(0, PT // STEP, bgroup, 0)


def _sc_gather1(*args):
    if "gather1" not in _sc_cache:
        _sc_cache["gather1"] = pl.kernel(
            _sc_gather1_body,
            out_type=jax.ShapeDtypeStruct((EP, ND), jnp.float32),
            mesh=_mesh(),
            scratch_types=[
                pltpu.VMEM_SHARED((N, ND), jnp.float32),
                pltpu.VMEM((KI, 128), jnp.int32),
                pltpu.VMEM((2, 128, ND), jnp.float32),
                pltpu.SemaphoreType.DMA,
                pltpu.SemaphoreType.DMA,
                pltpu.SemaphoreType.DMA,
                pltpu.SemaphoreType.DMA,
            ],
        )
    return _sc_cache["gather1"](*args)


def _sc_gather2(*args):
    if "gather2" not in _sc_cache:
        _sc_cache["gather2"] = pl.kernel(
            _sc_gather2_body,
            out_type=(
                jax.ShapeDtypeStruct((EP, ND), jnp.float32),
                jax.ShapeDtypeStruct((EP,), jnp.int32),
            ),
            mesh=_mesh(),
            scratch_types=[
                pltpu.VMEM_SHARED((N, ND), jnp.float32),
                pltpu.VMEM_SHARED((N + 8,), jnp.int32),
                pltpu.VMEM((KI, 128), jnp.int32),
                pltpu.VMEM((2, 128, ND), jnp.float32),
                pltpu.VMEM((STEP,), jnp.int32),
                pltpu.SemaphoreType.DMA,
                pltpu.SemaphoreType.DMA,
                pltpu.SemaphoreType.DMA,
                pltpu.SemaphoreType.DMA,
            ],
        )
    return _sc_cache["gather2"](*args)


def _sc_scatter_body(msg_hbm, dst2, zeros_hbm, out_hbm, acc, idx_v, rows_v,
                     sem):
    c = lax.axis_index("c")
    s = lax.axis_index("s")
    wid = s * NC + c
    rowbase = wid * (PT // 128)
    ebase = wid * PT

    pltpu.sync_copy(zeros_hbm, acc.at[pl.ds(s * RPT, RPT)])
    plsc.subcore_barrier()

    def step(o, _):
        pltpu.sync_copy(dst2.at[pl.ds(rowbase + o * KI, KI)], idx_v)
        for h in range(4):
            pltpu.sync_copy(
                msg_hbm.at[pl.ds(ebase + o * STEP + h * QTR, QTR)], rows_v
            )
            for j in range(2):
                pltpu.sync_copy(
                    rows_v.at[pl.ds(j * 128, 128)],
                    acc.at[idx_v.at[h * 2 + j]],
                    add=True,
                )
        return 0

    lax.fori_loop(0, PT // STEP, step, 0)
    plsc.subcore_barrier()
    pltpu.sync_copy(
        acc.at[pl.ds(s * RPT, RPT)], out_hbm.at[c, pl.ds(s * RPT, RPT)]
    )


def _sc_scatter(*args):
    if "scatter" not in _sc_cache:
        _sc_cache["scatter"] = pl.kernel(
            _sc_scatter_body,
            out_type=jax.ShapeDtypeStruct((NC, ACCR, ND), jnp.float32),
            mesh=_mesh(),
            scratch_types=[
                pltpu.VMEM_SHARED((ACCR, ND), jnp.float32),
                pltpu.VMEM((KI, 128), jnp.int32),
                pltpu.VMEM((QTR, ND), jnp.float32),
                pltpu.SemaphoreType.DMA,
            ],
        )
    return _sc_cache["scatter"](*args)


def _msg_body(xs_ref, ea_ref, wm1x_ref, wm1e_ref, bm1_ref, wm2_ref, bm2_ref,
              we1s_ref, msg_ref, p_ref):
    f32 = jnp.float32
    xs = xs_ref[...]
    h1 = jax.nn.relu(
        jnp.dot(xs, wm1x_ref[...], preferred_element_type=f32)
        + jnp.dot(ea_ref[...], wm1e_ref[...], preferred_element_type=f32)
        + bm1_ref[...]
    )
    msg_ref[...] = (
        jnp.dot(h1, wm2_ref[...], preferred_element_type=f32) + bm2_ref[...]
    )
    p_ref[...] = jnp.dot(xs, we1s_ref[...], preferred_element_type=f32)


def _eu_body(bsrc_ref, p_ref, xd_ref, ea_ref, g_ref, we1e_ref,
             we1d_ref, we1g_ref, be1_ref, we2_ref, be2_ref, eu_ref,
             agge_ref):
    i = pl.program_id(0)
    f32 = jnp.float32
    ea = ea_ref[...]
    b = bsrc_ref[0, 0, :]
    onehot = (
        b[:, None] == lax.broadcasted_iota(jnp.int32, (BE, B), 1)
    ).astype(f32)
    geff = jnp.dot(g_ref[...], we1g_ref[...], preferred_element_type=f32)
    h2 = jax.nn.relu(
        jnp.dot(ea, we1e_ref[...], preferred_element_type=f32)
        + p_ref[...]
        + jnp.dot(xd_ref[...], we1d_ref[...], preferred_element_type=f32)
        + jnp.dot(onehot, geff, preferred_element_type=f32)
        + be1_ref[...]
    )
    eu = jnp.dot(h2, we2_ref[...], preferred_element_type=f32) + be2_ref[...]
    eu_ref[...] = eu

    @pl.when(i == 0)
    def _():
        agge_ref[...] = jnp.zeros_like(agge_ref)

    agge_ref[...] += lax.dot_general(
        onehot, eu, (((0,), (0,)), ((), ())), preferred_element_type=f32
    )


def _node_body(batch_ref, x_ref, aggp_ref, g_ref, wn1x_ref, wn1m_ref,
               wn1g_ref, bn1_ref, wn2_ref, bn2_ref, xu_ref, aggn_ref):
    i = pl.program_id(0)
    f32 = jnp.float32
    x = x_ref[...]
    agg = aggp_ref[0] + aggp_ref[1]
    b = batch_ref[0, 0, :]
    onehot = (
        b[:, None] == lax.broadcasted_iota(jnp.int32, (BN, B), 1)
    ).astype(f32)
    gg = jnp.dot(g_ref[...], wn1g_ref[...], preferred_element_type=f32)
    h = jax.nn.relu(
        jnp.dot(x, wn1x_ref[...], preferred_element_type=f32)
        + jnp.dot(agg, wn1m_ref[...], preferred_element_type=f32)
        + jnp.dot(onehot, gg, preferred_element_type=f32)
        + bn1_ref[...]
    )
    xu = jnp.dot(h, wn2_ref[...], preferred_element_type=f32) + bn2_ref[...]
    xu_ref[...] = xu

    @pl.when(i == 0)
    def _():
        aggn_ref[...] = jnp.zeros_like(aggn_ref)

    aggn_ref[...] += lax.dot_general(
        onehot, xu, (((0,), (0,)), ((), ())), preferred_element_type=f32
    )


def _global_body(g_ref, an_ref, ae_ref, wg1g_ref, wg1n_ref, wg1e_ref,
                 bg1_ref, wg2_ref, bg2_ref, gu_ref):
    f32 = jnp.float32
    h = jax.nn.relu(
        jnp.dot(g_ref[...], wg1g_ref[...], preferred_element_type=f32)
        + jnp.dot(an_ref[...], wg1n_ref[...], preferred_element_type=f32)
        + jnp.dot(ae_ref[...], wg1e_ref[...], preferred_element_type=f32)
        + bg1_ref[...]
    )
    gu_ref[...] = jnp.dot(h, wg2_ref[...], preferred_element_type=f32) + bg2_ref[...]


def _full(shape):
    return pl.BlockSpec(shape, lambda i: (0,) * len(shape))


def kernel(x, edge_index, edge_attr, global_context_vector, batch, Wm1, bm1,
           Wm2, bm2, Wn1, bn1, Wn2, bn2, We1, be1, We2, be2, Wg1, bg1, Wg2,
           bg2):
    f32 = jnp.float32
    i32 = jnp.int32
    src = edge_index[0]
    dst = edge_index[1]
    padn = EP - E

    src2 = jnp.concatenate([src, jnp.zeros((padn,), i32)]).reshape(-1, 128)
    dstg2 = jnp.concatenate([dst, jnp.zeros((padn,), i32)]).reshape(-1, 128)
    srcb2 = jnp.concatenate([src, jnp.full((padn,), N, i32)]).reshape(-1, 128)
    dstp2 = jnp.concatenate([dst, jnp.full((padn,), N, i32)]).reshape(-1, 128)
    batch_ext = jnp.concatenate([batch, jnp.full((8,), B, i32)])

    xsrc = _sc_gather1(x, src2)
    xdst, bsrc = _sc_gather2(x, dstg2, srcb2, batch_ext)
    bsrc3 = bsrc.reshape(GE, 1, BE)

    g = global_context_vector
    bm1r = bm1.reshape(1, ND)
    bm2r = bm2.reshape(1, ND)
    be1r = be1.reshape(1, ED)
    be2r = be2.reshape(1, ED)
    bn1r = bn1.reshape(1, ND)
    bn2r = bn2.reshape(1, ND)
    bg1r = bg1.reshape(1, GD)
    bg2r = bg2.reshape(1, GD)

    def _clamp(i):
        return jnp.minimum(i, REAL - 1)

    msg, pxs = pl.pallas_call(
        _msg_body,
        grid=(GE,),
        in_specs=[
            pl.BlockSpec((BE, ND), lambda i: (i, 0)),
            pl.BlockSpec((BE, ED), lambda i: (_clamp(i), 0)),
            _full((ND, ND)), _full((ED, ND)), _full((1, ND)),
            _full((ND, ND)), _full((1, ND)),
            _full((ND, ED)),
        ],
        out_specs=[
            pl.BlockSpec((BE, ND), lambda i: (i, 0)),
            pl.BlockSpec((BE, ED), lambda i: (i, 0)),
        ],
        out_shape=[
            jax.ShapeDtypeStruct((EP, ND), f32),
            jax.ShapeDtypeStruct((EP, ED), f32),
        ],
    )(xsrc, edge_attr, Wm1[:ND], Wm1[ND:], bm1r, Wm2, bm2r,
      We1[ED:ED + ND])

    zeros_blk = jnp.zeros((RPT, ND), f32)
    aggp = _sc_scatter(msg, dstp2, zeros_blk)

    def _rot(i):
        return (i + REAL) % GE

    def _rotc(i):
        return jnp.minimum(_rot(i), REAL - 1)

    eu8, agg_e = pl.pallas_call(
        _eu_body,
        grid=(GE,),
        in_specs=[
            pl.BlockSpec((1, 1, BE), lambda i: (_rot(i), 0, 0)),
            pl.BlockSpec((BE, ED), lambda i: (_rot(i), 0)),
            pl.BlockSpec((BE, ND), lambda i: (_rot(i), 0)),
            pl.BlockSpec((BE, ED), lambda i: (_rotc(i), 0)),
            _full((B, GD)),
            _full((ED, ED)), _full((ND, ED)),
            _full((GD, ED)), _full((1, ED)),
            _full((ED, ED)), _full((1, ED)),
        ],
        out_specs=[
            pl.BlockSpec((BE, ED), lambda i: (_rotc(i), 0)),
            pl.BlockSpec((B, ED), lambda i: (0, 0)),
        ],
        out_shape=[
            jax.ShapeDtypeStruct((E, ED), f32),
            jax.ShapeDtypeStruct((B, ED), f32),
        ],
    )(bsrc3, pxs, xdst, edge_attr, g,
      We1[:ED], We1[ED + ND:ED + 2 * ND],
      We1[ED + 2 * ND:], be1r, We2, be2r)
    eu = eu8

    batch3 = batch.reshape(GN, 1, BN)
    xu, agg_n = pl.pallas_call(
        _node_body,
        grid=(GN,),
        in_specs=[
            pl.BlockSpec((1, 1, BN), lambda i: (i, 0, 0)),
            pl.BlockSpec((BN, ND), lambda i: (i, 0)),
            pl.BlockSpec((NC, BN, ND), lambda i: (0, i, 0)),
            _full((B, GD)),
            _full((ND, ND)), _full((ND, ND)), _full((GD, ND)), _full((1, ND)),
            _full((ND, ND)), _full((1, ND)),
        ],
        out_specs=[
            pl.BlockSpec((BN, ND), lambda i: (i, 0)),
            pl.BlockSpec((B, ND), lambda i: (0, 0)),
        ],
        out_shape=[
            jax.ShapeDtypeStruct((N, ND), f32),
            jax.ShapeDtypeStruct((B, ND), f32),
        ],
    )(batch3, x, aggp, g,
      Wn1[:ND], Wn1[ND:2 * ND], Wn1[2 * ND:], bn1r, Wn2, bn2r)

    gu = pl.pallas_call(
        _global_body,
        grid=(1,),
        in_specs=[
            _full((B, GD)), _full((B, ND)), _full((B, ED)),
            _full((GD, GD)), _full((ND, GD)), _full((ED, GD)), _full((1, GD)),
            _full((GD, GD)), _full((1, GD)),
        ],
        out_specs=pl.BlockSpec((B, GD), lambda i: (0, 0)),
        out_shape=jax.ShapeDtypeStruct((B, GD), f32),
    )(g, agg_n, agg_e,
      Wg1[:GD], Wg1[GD:GD + ND], Wg1[GD + ND:], bg1r, Wg2, bg2r)

    return (xu, eu, gu)


# speedup vs baseline: 1.0682x; 1.0682x over previous
import functools

import jax
import jax.numpy as jnp
from jax import lax
from jax.experimental import pallas as pl
from jax.experimental.pallas import tpu as pltpu
from jax.experimental.pallas import tpu_sc as plsc

N = 10000
E = 320000
B = 64
ND = 128
ED = 16
GD = 64

NC = 2
NS = 16
NW = NC * NS

PT = 10240
EP = NW * PT
KI = 8
STEP = KI * 128
HALF = STEP // 2
QTR = STEP // 4
ACCR = 10240
RPT = ACCR // NS

BE = 2560
GE = EP // BE
REAL = E // BE
BN = 2000
GN = N // BN

_sc_cache = {}


def _mesh():
    return plsc.VectorSubcoreMesh(
        core_axis_name="c", subcore_axis_name="s",
        num_cores=NC, num_subcores=NS,
    )


def _stage_x(x_hbm, xsp, s):
    @pl.when(s < NS - 1)
    def _():
        pltpu.sync_copy(x_hbm.at[pl.ds(s * 640, 640)],
                        xsp.at[pl.ds(s * 640, 640)])

    @pl.when(s == NS - 1)
    def _():
        pltpu.sync_copy(x_hbm.at[pl.ds(9600, 400)],
                        xsp.at[pl.ds(9600, 400)])


def _row_phase(idx2, out_hbm, xsp, idx_v, rows_v, gsem, osem, rowbase, ebase):
    def group(g, _):
        pltpu.sync_copy(idx2.at[pl.ds(rowbase + g * KI, KI)], idx_v)
        gd = [None, None]
        outs = [None, None]
        for c8 in range(KI):
            b = c8 % 2
            if outs[b] is not None:
                outs[b].wait()
            gd[b] = pltpu.async_copy(
                xsp.at[idx_v.at[c8]], rows_v.at[b], gsem[b]
            )
            if c8 >= 1:
                pb = 1 - b
                gd[pb].wait()
                outs[pb] = pltpu.async_copy(
                    rows_v.at[pb],
                    out_hbm.at[pl.ds(ebase + g * STEP + (c8 - 1) * 128, 128)],
                    osem[pb],
                )
        lb = (KI - 1) % 2
        gd[lb].wait()
        outs[lb] = pltpu.async_copy(
            rows_v.at[lb],
            out_hbm.at[pl.ds(ebase + g * STEP + (KI - 1) * 128, 128)],
            osem[lb],
        )
        outs[0].wait()
        outs[1].wait()
        return 0

    lax.fori_loop(0, PT // STEP, group, 0)


def _sc_gather_body(x_hbm, src2, dst2, srcb2, batch_ext, xsrc_out, xdst_out,
                    bsrc_out, xsp, bsp, idx_v, rows_v, bvals_v,
                    gsem0, gsem1, osem0, osem1):
    c = lax.axis_index("c")
    s = lax.axis_index("s")
    wid = s * NC + c
    rowbase = wid * (PT // 128)
    ebase = wid * PT
    _stage_x(x_hbm, xsp, s)

    @pl.when(s == 0)
    def _():
        pltpu.sync_copy(batch_ext, bsp)

    plsc.subcore_barrier()
    _row_phase(src2, xsrc_out, xsp, idx_v, rows_v, [gsem0, gsem1],
               [osem0, osem1], rowbase, ebase)
    _row_phase(dst2, xdst_out, xsp, idx_v, rows_v, [gsem0, gsem1],
               [osem0, osem1], rowbase, ebase)

    def bgroup(g, _):
        pltpu.sync_copy(srcb2.at[pl.ds(rowbase + g * KI, KI)], idx_v)
        descs = [
            pltpu.async_copy(
                bsp.at[idx_v.at[j]],
                bvals_v.at[pl.ds(j * 128, 128)],
                gsem0,
            )
            for j in range(KI)
        ]
        for d in descs:
            d.wait()
        pltpu.sync_copy(bvals_v, bsrc_out.at[pl.ds(ebase + g * STEP, STEP)])
        return 0

    lax.fori_loop(0, PT // STEP, bgroup, 0)


def _sc_gather(*args):
    if "gather" not in _sc_cache:
        _sc_cache["gather"] = pl.kernel(
            _sc_gather_body,
            out_type=(
                jax.ShapeDtypeStruct((EP, ND), jnp.float32),
                jax.ShapeDtypeStruct((EP, ND), jnp.float32),
                jax.ShapeDtypeStruct((EP,), jnp.int32),
            ),
            mesh=_mesh(),
            scratch_types=[
                pltpu.VMEM_SHARED((N, ND), jnp.float32),
                pltpu.VMEM_SHARED((N + 8,), jnp.int32),
                pltpu.VMEM((KI, 128), jnp.int32),
                pltpu.VMEM((2, 128, ND), jnp.float32),
                pltpu.VMEM((STEP,), jnp.int32),
                pltpu.SemaphoreType.DMA,
                pltpu.SemaphoreType.DMA,
                pltpu.SemaphoreType.DMA,
                pltpu.SemaphoreType.DMA,
            ],
        )
    return _sc_cache["gather"](*args)


def _sc_scatter_body(msg_hbm, dst2, zeros_hbm, out_hbm, acc, idx_v, rows_v,
                     sem):
    c = lax.axis_index("c")
    s = lax.axis_index("s")
    wid = s * NC + c
    rowbase = wid * (PT // 128)
    ebase = wid * PT

    pltpu.sync_copy(zeros_hbm, acc.at[pl.ds(s * RPT, RPT)])
    plsc.subcore_barrier()

    def step(o, _):
        pltpu.sync_copy(dst2.at[pl.ds(rowbase + o * KI, KI)], idx_v)
        for h in range(4):
            pltpu.sync_copy(
                msg_hbm.at[pl.ds(ebase + o * STEP + h * QTR, QTR)], rows_v
            )
            for j in range(2):
                pltpu.sync_copy(
                    rows_v.at[pl.ds(j * 128, 128)],
                    acc.at[idx_v.at[h * 2 + j]],
                    add=True,
                )
        return 0

    lax.fori_loop(0, PT // STEP, step, 0)
    plsc.subcore_barrier()
    pltpu.sync_copy(
        acc.at[pl.ds(s * RPT, RPT)], out_hbm.at[c, pl.ds(s * RPT, RPT)]
    )


def _sc_scatter(*args):
    if "scatter" not in _sc_cache:
        _sc_cache["scatter"] = pl.kernel(
            _sc_scatter_body,
            out_type=jax.ShapeDtypeStruct((NC, ACCR, ND), jnp.float32),
            mesh=_mesh(),
            scratch_types=[
                pltpu.VMEM_SHARED((ACCR, ND), jnp.float32),
                pltpu.VMEM((KI, 128), jnp.int32),
                pltpu.VMEM((QTR, ND), jnp.float32),
                pltpu.SemaphoreType.DMA,
            ],
        )
    return _sc_cache["scatter"](*args)


def _edge_body(bsrc_ref, xs_ref, xd_ref, ea_ref, g_ref, wm1x_ref, wm1e_ref,
               bm1_ref, wm2_ref, bm2_ref, we1e_ref, we1s_ref, we1d_ref,
               we1g_ref, be1_ref, we2_ref, be2_ref, msg_ref, eu_ref,
               agge_ref):
    i = pl.program_id(0)
    xs = xs_ref[...]
    xd = xd_ref[...]
    ea = ea_ref[...]
    f32 = jnp.float32

    h1 = jax.nn.relu(
        jnp.dot(xs, wm1x_ref[...], preferred_element_type=f32)
        + jnp.dot(ea, wm1e_ref[...], preferred_element_type=f32)
        + bm1_ref[...]
    )
    msg_ref[...] = (
        jnp.dot(h1, wm2_ref[...], preferred_element_type=f32) + bm2_ref[...]
    )

    b = bsrc_ref[0, 0, :]
    onehot = (
        b[:, None] == lax.broadcasted_iota(jnp.int32, (BE, B), 1)
    ).astype(f32)
    geff = jnp.dot(g_ref[...], we1g_ref[...], preferred_element_type=f32)
    h2 = jax.nn.relu(
        jnp.dot(ea, we1e_ref[...], preferred_element_type=f32)
        + jnp.dot(xs, we1s_ref[...], preferred_element_type=f32)
        + jnp.dot(xd, we1d_ref[...], preferred_element_type=f32)
        + jnp.dot(onehot, geff, preferred_element_type=f32)
        + be1_ref[...]
    )
    eu = jnp.dot(h2, we2_ref[...], preferred_element_type=f32) + be2_ref[...]
    eu_ref[...] = eu

    @pl.when(i == 0)
    def _():
        agge_ref[...] = jnp.zeros_like(agge_ref)

    agge_ref[...] += lax.dot_general(
        onehot, eu, (((0,), (0,)), ((), ())), preferred_element_type=f32
    )


def _node_body(batch_ref, x_ref, aggp_ref, g_ref, wn1x_ref, wn1m_ref,
               wn1g_ref, bn1_ref, wn2_ref, bn2_ref, xu_ref, aggn_ref):
    i = pl.program_id(0)
    f32 = jnp.float32
    x = x_ref[...]
    agg = aggp_ref[0] + aggp_ref[1]
    b = batch_ref[0, 0, :]
    onehot = (
        b[:, None] == lax.broadcasted_iota(jnp.int32, (BN, B), 1)
    ).astype(f32)
    gg = jnp.dot(g_ref[...], wn1g_ref[...], preferred_element_type=f32)
    h = jax.nn.relu(
        jnp.dot(x, wn1x_ref[...], preferred_element_type=f32)
        + jnp.dot(agg, wn1m_ref[...], preferred_element_type=f32)
        + jnp.dot(onehot, gg, preferred_element_type=f32)
        + bn1_ref[...]
    )
    xu = jnp.dot(h, wn2_ref[...], preferred_element_type=f32) + bn2_ref[...]
    xu_ref[...] = xu

    @pl.when(i == 0)
    def _():
        aggn_ref[...] = jnp.zeros_like(aggn_ref)

    aggn_ref[...] += lax.dot_general(
        onehot, xu, (((0,), (0,)), ((), ())), preferred_element_type=f32
    )


def _global_body(g_ref, an_ref, ae_ref, wg1g_ref, wg1n_ref, wg1e_ref,
                 bg1_ref, wg2_ref, bg2_ref, gu_ref):
    f32 = jnp.float32
    h = jax.nn.relu(
        jnp.dot(g_ref[...], wg1g_ref[...], preferred_element_type=f32)
        + jnp.dot(an_ref[...], wg1n_ref[...], preferred_element_type=f32)
        + jnp.dot(ae_ref[...], wg1e_ref[...], preferred_element_type=f32)
        + bg1_ref[...]
    )
    gu_ref[...] = jnp.dot(h, wg2_ref[...], preferred_element_type=f32) + bg2_ref[...]


def _full(shape):
    return pl.BlockSpec(shape, lambda i: (0,) * len(shape))


def kernel(x, edge_index, edge_attr, global_context_vector, batch, Wm1, bm1,
           Wm2, bm2, Wn1, bn1, Wn2, bn2, We1, be1, We2, be2, Wg1, bg1, Wg2,
           bg2):
    f32 = jnp.float32
    i32 = jnp.int32
    src = edge_index[0]
    dst = edge_index[1]
    padn = EP - E

    src2 = jnp.concatenate([src, jnp.zeros((padn,), i32)]).reshape(-1, 128)
    dstg2 = jnp.concatenate([dst, jnp.zeros((padn,), i32)]).reshape(-1, 128)
    srcb2 = jnp.concatenate([src, jnp.full((padn,), N, i32)]).reshape(-1, 128)
    dstp2 = jnp.concatenate([dst, jnp.full((padn,), N, i32)]).reshape(-1, 128)
    batch_ext = jnp.concatenate([batch, jnp.full((8,), B, i32)])

    xsrc, xdst, bsrc = _sc_gather(x, src2, dstg2, srcb2, batch_ext)
    bsrc3 = bsrc.reshape(GE, 1, BE)

    g = global_context_vector
    bm1r = bm1.reshape(1, ND)
    bm2r = bm2.reshape(1, ND)
    be1r = be1.reshape(1, ED)
    be2r = be2.reshape(1, ED)
    bn1r = bn1.reshape(1, ND)
    bn2r = bn2.reshape(1, ND)
    bg1r = bg1.reshape(1, GD)
    bg2r = bg2.reshape(1, GD)

    def _rot(i):
        return (i + REAL) % GE

    def _rotc(i):
        return jnp.minimum(_rot(i), REAL - 1)

    msg, eu, agg_e = pl.pallas_call(
        _edge_body,
        grid=(GE,),
        in_specs=[
            pl.BlockSpec((1, 1, BE), lambda i: (_rot(i), 0, 0)),
            pl.BlockSpec((BE, ND), lambda i: (_rot(i), 0)),
            pl.BlockSpec((BE, ND), lambda i: (_rot(i), 0)),
            pl.BlockSpec((BE, ED), lambda i: (_rotc(i), 0)),
            _full((B, GD)),
            _full((ND, ND)), _full((ED, ND)), _full((1, ND)),
            _full((ND, ND)), _full((1, ND)),
            _full((ED, ED)), _full((ND, ED)), _full((ND, ED)),
            _full((GD, ED)), _full((1, ED)),
            _full((ED, ED)), _full((1, ED)),
        ],
        out_specs=[
            pl.BlockSpec((BE, ND), lambda i: (_rot(i), 0)),
            pl.BlockSpec((BE, ED), lambda i: (_rotc(i), 0)),
            pl.BlockSpec((B, ED), lambda i: (0, 0)),
        ],
        out_shape=[
            jax.ShapeDtypeStruct((EP, ND), f32),
            jax.ShapeDtypeStruct((E, ED), f32),
            jax.ShapeDtypeStruct((B, ED), f32),
        ],
    )(bsrc3, xsrc, xdst, edge_attr, g,
      Wm1[:ND], Wm1[ND:], bm1r, Wm2, bm2r,
      We1[:ED], We1[ED:ED + ND], We1[ED + ND:ED + 2 * ND],
      We1[ED + 2 * ND:], be1r, We2, be2r)

    zeros_blk = jnp.zeros((RPT, ND), f32)
    aggp = _sc_scatter(msg, dstp2, zeros_blk)

    batch3 = batch.reshape(GN, 1, BN)
    xu, agg_n = pl.pallas_call(
        _node_body,
        grid=(GN,),
        in_specs=[
            pl.BlockSpec((1, 1, BN), lambda i: (i, 0, 0)),
            pl.BlockSpec((BN, ND), lambda i: (i, 0)),
            pl.BlockSpec((NC, BN, ND), lambda i: (0, i, 0)),
            _full((B, GD)),
            _full((ND, ND)), _full((ND, ND)), _full((GD, ND)), _full((1, ND)),
            _full((ND, ND)), _full((1, ND)),
        ],
        out_specs=[
            pl.BlockSpec((BN, ND), lambda i: (i, 0)),
            pl.BlockSpec((B, ND), lambda i: (0, 0)),
        ],
        out_shape=[
            jax.ShapeDtypeStruct((N, ND), f32),
            jax.ShapeDtypeStruct((B, ND), f32),
        ],
    )(batch3, x, aggp, g,
      Wn1[:ND], Wn1[ND:2 * ND], Wn1[2 * ND:], bn1r, Wn2, bn2r)

    gu = pl.pallas_call(
        _global_body,
        grid=(1,),
        in_specs=[
            _full((B, GD)), _full((B, ND)), _full((B, ED)),
            _full((GD, GD)), _full((ND, GD)), _full((ED, GD)), _full((1, GD)),
            _full((GD, GD)), _full((1, GD)),
        ],
        out_specs=pl.BlockSpec((B, GD), lambda i: (0, 0)),
        out_shape=jax.ShapeDtypeStruct((B, GD), f32),
    )(g, agg_n, agg_e,
      Wg1[:GD], Wg1[GD:GD + ND], Wg1[GD + ND:], bg1r, Wg2, bg2r)

    return (xu, eu, gu)
